# Initial kernel scaffold; baseline (speedup 1.0000x reference)
import jax
import jax.numpy as jnp
from jax.experimental import pallas as pl


def kernel(z, lambda_pairs, codebook):
    raise NotImplementedError("write your pallas kernel here")


# speedup vs baseline: 1.6674x; 1.6674x over previous
import functools

import jax
import jax.numpy as jnp
from jax.experimental import pallas as pl

NUM_EMBEDDINGS = 1024
EMBEDDING_DIM = 64
COMMITMENT_COST = 0.25

_BLOCK_ROWS = 1024


def _fused_kernel(x_ref, cbp_ref, lam_ref, a2_ref, zq_ref, idx_ref, loss_ref):
    i = pl.program_id(0)
    x = x_ref[...]
    cb = cbp_ref[:, 0:EMBEDDING_DIM]
    cbn = cbp_ref[:, EMBEDDING_DIM:2 * EMBEDDING_DIM]
    lam = lam_ref[...]

    dcb = (1.0 - lam) * cb + lam * cbn
    b2 = jnp.sum(dcb * dcb, axis=1)
    col = jax.lax.broadcasted_iota(jnp.int32, (1, NUM_EMBEDDINGS), 1)
    b2 = jnp.where(col[0] == NUM_EMBEDDINGS - 1, jnp.float32(1e30), b2)

    a2 = a2_ref[...]
    m = jax.lax.dot_general(
        x, dcb, (((1,), (1,)), ((), ())),
        preferred_element_type=jnp.float32)
    scores = jnp.sqrt(jnp.maximum((a2 + b2[None, :]) - 2.0 * m, 0.0))

    mn = jnp.min(scores, axis=1, keepdims=True)
    cols = jax.lax.broadcasted_iota(jnp.int32, scores.shape, 1)
    idx = jnp.min(jnp.where(scores == mn, cols, NUM_EMBEDDINGS),
                  axis=1, keepdims=True)
    idx_ref[...] = idx

    onehot = (cols == idx).astype(jnp.float32)
    g = jax.lax.dot_general(
        onehot, cbp_ref[...], (((1,), (0,)), ((), ())),
        preferred_element_type=jnp.float32,
        precision=jax.lax.Precision.HIGHEST)
    c_i = g[:, 0:EMBEDDING_DIM]
    c_ip1 = g[:, EMBEDDING_DIM:2 * EMBEDDING_DIM]
    lam_i = g[:, 2 * EMBEDDING_DIM:2 * EMBEDDING_DIM + 1]

    d_i = c_i - x
    d_ip1 = c_ip1 - x
    n_i = jnp.sqrt(jnp.sum(d_i * d_i, axis=1, keepdims=True))
    n_ip1 = jnp.sqrt(jnp.sum(d_ip1 * d_ip1, axis=1, keepdims=True))
    s_i = n_i / (n_i + 1e-8)
    s_ip1 = n_ip1 / (n_ip1 + 1e-8)
    zq_ref[...] = x + (1.0 - lam_i) * d_i * s_i + lam_i * d_ip1 * s_ip1

    dt = (1.0 - lam_i) * c_i + lam_i * c_ip1
    r = dt - x
    part = (jnp.sum(r * r) * jnp.float32(
        (1.0 + COMMITMENT_COST) / (16 * 576 * EMBEDDING_DIM))).reshape(1, 1)

    @pl.when(i == 0)
    def _():
        loss_ref[...] = part

    @pl.when(i != 0)
    def _():
        loss_ref[...] += part


@functools.partial(jax.jit, static_argnames=("interpret",))
def kernel(z, lambda_pairs, codebook, interpret=False):
    n = z.shape[0] * z.shape[1]
    flat = z.reshape(n, EMBEDDING_DIM)
    cb_next = jnp.concatenate([codebook[1:], codebook[:1]], axis=0)
    lam_pad = jnp.concatenate(
        [lambda_pairs, jnp.zeros((1, 1), jnp.float32)], axis=0)
    cbp = jnp.concatenate([codebook, cb_next, lam_pad], axis=1)
    a2 = jnp.sum(flat ** 2, axis=1, keepdims=True)

    grid = n // _BLOCK_ROWS
    zq, idx, loss = pl.pallas_call(
        _fused_kernel,
        grid=(grid,),
        in_specs=[
            pl.BlockSpec((_BLOCK_ROWS, EMBEDDING_DIM), lambda i: (i, 0)),
            pl.BlockSpec((NUM_EMBEDDINGS, 2 * EMBEDDING_DIM + 1),
                         lambda i: (0, 0)),
            pl.BlockSpec((NUM_EMBEDDINGS, 1), lambda i: (0, 0)),
            pl.BlockSpec((_BLOCK_ROWS, 1), lambda i: (i, 0)),
        ],
        out_specs=[
            pl.BlockSpec((_BLOCK_ROWS, EMBEDDING_DIM), lambda i: (i, 0)),
            pl.BlockSpec((_BLOCK_ROWS, 1), lambda i: (i, 0)),
            pl.BlockSpec((1, 1), lambda i: (0, 0)),
        ],
        out_shape=[
            jax.ShapeDtypeStruct((n, EMBEDDING_DIM), jnp.float32),
            jax.ShapeDtypeStruct((n, 1), jnp.int32),
            jax.ShapeDtypeStruct((1, 1), jnp.float32),
        ],
        interpret=interpret,
    )(flat, cbp, lam_pad, a2)

    return (zq.reshape(z.shape), loss[0, 0],
            idx[:, 0].reshape(z.shape[:-1]))


# speedup vs baseline: 2.3753x; 1.4246x over previous
import functools

import jax
import jax.numpy as jnp
from jax.experimental import pallas as pl

NUM_EMBEDDINGS = 1024
EMBEDDING_DIM = 64
COMMITMENT_COST = 0.25

_BLOCK_ROWS = 1024


def _fused_kernel(x_ref, cbp_ref, lam_ref, a2_ref, zq_ref, idx_ref, loss_ref):
    i = pl.program_id(0)
    x = x_ref[...]
    cb = cbp_ref[:, 0:EMBEDDING_DIM]
    cbn = cbp_ref[:, EMBEDDING_DIM:2 * EMBEDDING_DIM]
    lam = lam_ref[...]

    dcb = (1.0 - lam) * cb + lam * cbn
    b2 = jnp.sum(dcb * dcb, axis=1)
    col = jax.lax.broadcasted_iota(jnp.int32, (1, NUM_EMBEDDINGS), 1)
    b2 = jnp.where(col[0] == NUM_EMBEDDINGS - 1, jnp.float32(1e30), b2)

    a2 = a2_ref[...]
    m = jax.lax.dot_general(
        x, dcb, (((1,), (1,)), ((), ())),
        preferred_element_type=jnp.float32)
    scores = jnp.sqrt(jnp.maximum((a2 + b2[None, :]) - 2.0 * m, 0.0))

    mn = jnp.min(scores, axis=1, keepdims=True)
    cols = jax.lax.broadcasted_iota(jnp.int32, scores.shape, 1)
    idx = jnp.min(jnp.where(scores == mn, cols, NUM_EMBEDDINGS),
                  axis=1, keepdims=True)
    idx_ref[...] = idx

    onehot = (cols == idx).astype(jnp.float32)
    g = jax.lax.dot_general(
        onehot, cbp_ref[...], (((1,), (0,)), ((), ())),
        preferred_element_type=jnp.float32)
    c_i = g[:, 0:EMBEDDING_DIM]
    c_ip1 = g[:, EMBEDDING_DIM:2 * EMBEDDING_DIM]
    lam_i = g[:, 2 * EMBEDDING_DIM:2 * EMBEDDING_DIM + 1]

    d_i = c_i - x
    d_ip1 = c_ip1 - x
    n_i = jnp.sqrt(jnp.sum(d_i * d_i, axis=1, keepdims=True))
    n_ip1 = jnp.sqrt(jnp.sum(d_ip1 * d_ip1, axis=1, keepdims=True))
    s_i = n_i / (n_i + 1e-8)
    s_ip1 = n_ip1 / (n_ip1 + 1e-8)
    zq_ref[...] = x + (1.0 - lam_i) * d_i * s_i + lam_i * d_ip1 * s_ip1

    dt = (1.0 - lam_i) * c_i + lam_i * c_ip1
    r = dt - x
    part = (jnp.sum(r * r) * jnp.float32(
        (1.0 + COMMITMENT_COST) / (16 * 576 * EMBEDDING_DIM))).reshape(1, 1)

    @pl.when(i == 0)
    def _():
        loss_ref[...] = part

    @pl.when(i != 0)
    def _():
        loss_ref[...] += part


@functools.partial(jax.jit, static_argnames=("interpret",))
def kernel(z, lambda_pairs, codebook, interpret=False):
    n = z.shape[0] * z.shape[1]
    flat = z.reshape(n, EMBEDDING_DIM)
    cb_next = jnp.concatenate([codebook[1:], codebook[:1]], axis=0)
    lam_pad = jnp.concatenate(
        [lambda_pairs, jnp.zeros((1, 1), jnp.float32)], axis=0)
    cbp = jnp.concatenate([codebook, cb_next, lam_pad], axis=1)
    a2 = jnp.sum(flat ** 2, axis=1, keepdims=True)

    grid = n // _BLOCK_ROWS
    zq, idx, loss = pl.pallas_call(
        _fused_kernel,
        grid=(grid,),
        in_specs=[
            pl.BlockSpec((_BLOCK_ROWS, EMBEDDING_DIM), lambda i: (i, 0)),
            pl.BlockSpec((NUM_EMBEDDINGS, 2 * EMBEDDING_DIM + 1),
                         lambda i: (0, 0)),
            pl.BlockSpec((NUM_EMBEDDINGS, 1), lambda i: (0, 0)),
            pl.BlockSpec((_BLOCK_ROWS, 1), lambda i: (i, 0)),
        ],
        out_specs=[
            pl.BlockSpec((_BLOCK_ROWS, EMBEDDING_DIM), lambda i: (i, 0)),
            pl.BlockSpec((_BLOCK_ROWS, 1), lambda i: (i, 0)),
            pl.BlockSpec((1, 1), lambda i: (0, 0)),
        ],
        out_shape=[
            jax.ShapeDtypeStruct((n, EMBEDDING_DIM), jnp.float32),
            jax.ShapeDtypeStruct((n, 1), jnp.int32),
            jax.ShapeDtypeStruct((1, 1), jnp.float32),
        ],
        interpret=interpret,
    )(flat, cbp, lam_pad, a2)

    return (zq.reshape(z.shape), loss[0, 0],
            idx[:, 0].reshape(z.shape[:-1]))
